# baseline (speedup 1.0000x reference)
import jax
import jax.numpy as jnp
from jax import lax
from jax.experimental import pallas as pl
from jax.experimental.pallas import tpu as pltpu
from jax.experimental.pallas import tpu_sc as plsc

_POS_HID = 32
_ATTN_HID = 12
_B = 4
_N = 512
_L = 16
_NC = 2
_NS = 16
_TPB = (_NC * _NS) // _B
_PPT = _N // _TPB

_OQKV = 0
_OW1 = 27
_OB1 = 123
_OW2 = 155
_OB2 = 251
_OA1 = 254
_OAB1 = 290
_OA2 = 302
_OAB2 = 338
_PER_LAYER = 341
_OLW = 3 * _PER_LAYER
_OLB = _OLW + 6
_NW = _OLB + 2
_NWPAD = ((_NW + _L - 1) // _L) * _L


def _pack_weights(params):
  segs = []
  for ln in ('l1', 'l2', 'l3'):
    p = params[ln]
    segs += [p['qkv'].reshape(-1), p['pos_w1'].reshape(-1), p['pos_b1'],
             p['pos_w2'].reshape(-1), p['pos_b2'], p['attn_w1'].reshape(-1),
             p['attn_b1'], p['attn_w2'].reshape(-1), p['attn_b2']]
  segs += [params['lin_w'].reshape(-1), params['lin_b'],
           jnp.zeros((_NWPAD - _NW,), jnp.float32)]
  w = jnp.concatenate(segs).astype(jnp.float32)
  return jnp.repeat(w[:_NW], _L), w


def _sc_body(pos_hbm, x_hbm, w_hbm, wp_hbm, out_hbm,
             posb, xb, qkvb, idxb, rpb, outb, xnb, fb, wv, wpb, xsh):
  c = lax.axis_index('c')
  s = lax.axis_index('s')
  bl = s // _TPB
  b = c * 2 + bl
  ch = s % _TPB
  base = ch * _PPT

  pltpu.sync_copy(pos_hbm.at[b], posb)
  pltpu.sync_copy(x_hbm.at[b], xb)
  pltpu.sync_copy(w_hbm, wv)
  pltpu.sync_copy(wp_hbm, wpb)

  iota = lax.iota(jnp.int32, _L)

  def wsp(j):
    return wv[pl.ds(_L * j, _L)]

  wregs = {}

  def wbc(j):
    v = j // _L
    if v not in wregs:
      wregs[v] = wpb[pl.ds(v * _L, _L)]
    return jnp.full((_L,), wregs[v][j % _L], jnp.float32)

  _tgl = [False]

  def wgt(j):
    _tgl[0] = not _tgl[0]
    return wsp(j) if _tgl[0] else wbc(j)

  _Q = 4
  _G = _PPT // _Q

  @plsc.parallel_loop(0, _G)
  def _topk_loop(i):
    pts = []
    for q in range(_Q):
      gq = jnp.full((_L,), base + i + q * _G, jnp.int32)
      pts.append((plsc.load_gather(posb, [gq]),
                  plsc.load_gather(posb, [gq + _N]),
                  plsc.load_gather(posb, [gq + 2 * _N])))

    def chunk_body(j, carry):
      off = j * _L
      cx = posb[pl.ds(off, _L)]
      cy = posb[pl.ds(_N + off, _L)]
      cz = posb[pl.ds(2 * _N + off, _L)]
      ci = iota + off
      out = []
      for q in range(_Q):
        px, py, pz = pts[q]
        bk, bv = carry[2 * q], carry[2 * q + 1]
        dx = px - cx
        dy = py - cy
        dz = pz - cz
        d2 = dx * dx + dy * dy + dz * dz
        ck, cv = plsc.sort_key_val(d2, ci, descending=True)
        keep = bk <= ck
        mk = jnp.where(keep, bk, ck)
        mv = jnp.where(keep, bv, cv)
        nk, nv = plsc.sort_key_val(mk, mv)
        out += [nk, nv]
      return tuple(out)

    inf0 = jnp.full((_L,), jnp.inf, jnp.float32)
    iz = jnp.zeros((_L,), jnp.int32)
    res = lax.fori_loop(0, _N // _L, chunk_body, (inf0, iz) * _Q)

    for q in range(_Q):
      pi = i + q * _G
      px, py, pz = pts[q]
      bv = res[2 * q + 1]
      idxb[pl.ds(pi * _L, _L)] = bv
      gx = plsc.load_gather(posb, [bv])
      gy = plsc.load_gather(posb, [bv + _N])
      gz = plsc.load_gather(posb, [bv + 2 * _N])
      rpb[pl.ds(pi * _L, _L)] = px - gx
      rpb[pl.ds((_PPT + pi) * _L, _L)] = py - gy
      rpb[pl.ds((2 * _PPT + pi) * _L, _L)] = pz - gz

  for l in range(3):
    lb = l * _PER_LAYER

    @plsc.parallel_loop(0, _N // _L)
    def _qkv_loop(j, lb=lb):
      wregs.clear()
      off = j * _L
      x0 = xb[pl.ds(off, _L)]
      x1 = xb[pl.ds(_N + off, _L)]
      x2 = xb[pl.ds(2 * _N + off, _L)]
      for u in range(9):
        qkvb[pl.ds(u * _N + off, _L)] = (x0 * wgt(lb + _OQKV + u) +
                                         x1 * wgt(lb + _OQKV + 9 + u) +
                                         x2 * wgt(lb + _OQKV + 18 + u))

    @plsc.parallel_loop(0, _PPT // 2, unroll=4)
    def _pt_loop(i, lb=lb):
      wregs.clear()
      pts = []
      for pi in (i, i + _PPT // 2):
        gidx = jnp.full((_L,), base + pi, jnp.int32)
        nb = idxb[pl.ds(pi * _L, _L)]
        qx = plsc.load_gather(qkvb, [gidx])
        qy = plsc.load_gather(qkvb, [gidx + _N])
        qz = plsc.load_gather(qkvb, [gidx + 2 * _N])
        kx = plsc.load_gather(qkvb, [nb + 3 * _N])
        ky = plsc.load_gather(qkvb, [nb + 4 * _N])
        kz = plsc.load_gather(qkvb, [nb + 5 * _N])
        vx = plsc.load_gather(qkvb, [nb + 6 * _N])
        vy = plsc.load_gather(qkvb, [nb + 7 * _N])
        vz = plsc.load_gather(qkvb, [nb + 8 * _N])
        rx = rpb[pl.ds(pi * _L, _L)]
        ry = rpb[pl.ds((_PPT + pi) * _L, _L)]
        rz = rpb[pl.ds((2 * _PPT + pi) * _L, _L)]
        pts.append({'pi': pi, 'q': (qx, qy, qz), 'k': (kx, ky, kz),
                    'v': (vx, vy, vz), 'r': (rx, ry, rz)})

      pb = [wgt(lb + _OB2 + cc) for cc in range(3)]
      for p in pts:
        p['pe'] = list(pb)
      for u in range(_POS_HID):
        w1 = [wgt(lb + _OW1 + 32 * cc + u) for cc in range(3)]
        b1 = wgt(lb + _OB1 + u)
        w2 = [wgt(lb + _OW2 + u * 3 + cc) for cc in range(3)]
        for p in pts:
          rx, ry, rz = p['r']
          h = jnp.maximum(rx * w1[0] + ry * w1[1] + rz * w1[2] + b1, 0.0)
          for cc in range(3):
            p['pe'][cc] = p['pe'][cc] + h * w2[cc]

      ab = [wgt(lb + _OAB2 + cc) for cc in range(3)]
      for p in pts:
        p['s'] = [p['q'][cc] - p['k'][cc] + p['pe'][cc] for cc in range(3)]
        p['o'] = list(ab)
      for u in range(_ATTN_HID):
        a1 = [wgt(lb + _OA1 + 12 * cc + u) for cc in range(3)]
        ob1 = wgt(lb + _OAB1 + u)
        a2 = [wgt(lb + _OA2 + u * 3 + cc) for cc in range(3)]
        for p in pts:
          sx, sy, sz = p['s']
          g = jnp.maximum(sx * a1[0] + sy * a1[1] + sz * a1[2] + ob1, 0.0)
          for cc in range(3):
            p['o'][cc] = p['o'][cc] + g * a2[cc]

      for p in pts:
        outs = []
        for cc in range(3):
          e = jnp.exp(p['o'][cc])
          a = e / jnp.sum(e)
          outs.append(jnp.sum(a * (p['v'][cc] + p['pe'][cc])))
        vout = jnp.where(iota == 0, outs[0],
                         jnp.where(iota == 1, outs[1], outs[2]))
        plsc.store_scatter(outb, [jnp.minimum(iota, 2) * _PPT + p['pi']],
                           vout, mask=iota < 3)

    if l < 2:
      for t in range(_PPT // _L):
        for cc in range(3):
          z = outb[pl.ds(cc * _PPT + t * _L, _L)]
          xnb[pl.ds(cc * _PPT + t * _L, _L)] = 1.0 / (1.0 + jnp.exp(-z))
      for cc in range(3):
        pltpu.sync_copy(xnb.at[pl.ds(cc * _PPT, _PPT)],
                        xsh.at[bl, pl.ds(cc * _N + base, _PPT)])
      plsc.subcore_barrier()
      pltpu.sync_copy(xsh.at[bl], xb)
      plsc.subcore_barrier()
    else:
      for t in range(_PPT // _L):
        xs = []
        for cc in range(3):
          z = outb[pl.ds(cc * _PPT + t * _L, _L)]
          xs.append(1.0 / (1.0 + jnp.exp(-z)))
        u0 = (xs[0] * wsp(_OLW + 0) + xs[1] * wsp(_OLW + 2) +
              xs[2] * wsp(_OLW + 4) + wsp(_OLB + 0))
        u1 = (xs[0] * wsp(_OLW + 1) + xs[1] * wsp(_OLW + 3) +
              xs[2] * wsp(_OLW + 5) + wsp(_OLB + 1))
        m = jnp.maximum(u0, u1)
        e0 = jnp.exp(u0 - m)
        e1 = jnp.exp(u1 - m)
        tot = e0 + e1
        fb[pl.ds(t * _L, _L)] = e0 / tot
        fb[pl.ds(_PPT + t * _L, _L)] = e1 / tot
      for o in range(2):
        pltpu.sync_copy(fb.at[pl.ds(o * _PPT, _PPT)],
                        out_hbm.at[b, pl.ds(o * _N + base, _PPT)])


@jax.jit
def _sc_call(pos_t, x_t, wflat, wpack):
  mesh = plsc.VectorSubcoreMesh(core_axis_name='c', subcore_axis_name='s',
                                num_cores=_NC, num_subcores=_NS)
  return pl.kernel(
      _sc_body,
      out_type=jax.ShapeDtypeStruct((_B, 2 * _N), jnp.float32),
      mesh=mesh,
      scratch_types=[
          pltpu.VMEM((3 * _N,), jnp.float32),
          pltpu.VMEM((3 * _N,), jnp.float32),
          pltpu.VMEM((9 * _N,), jnp.float32),
          pltpu.VMEM((_PPT * _L,), jnp.int32),
          pltpu.VMEM((3 * _PPT * _L,), jnp.float32),
          pltpu.VMEM((3 * _PPT,), jnp.float32),
          pltpu.VMEM((3 * _PPT,), jnp.float32),
          pltpu.VMEM((2 * _PPT,), jnp.float32),
          pltpu.VMEM((_NW * _L,), jnp.float32),
          pltpu.VMEM((_NWPAD,), jnp.float32),
          pltpu.VMEM_SHARED((2, 3 * _N), jnp.float32),
      ],
      compiler_params=pltpu.CompilerParams(use_tc_tiling_on_sc=False,
                                           needs_layout_passes=False),
      name='pt_knn_sc',
  )(pos_t, x_t, wflat, wpack)


def kernel(feats, pos, mask, params):
  del mask
  pos_t = jnp.transpose(pos, (0, 2, 1)).reshape(_B, 3 * _N).astype(jnp.float32)
  x_t = jnp.transpose(feats, (0, 2, 1)).reshape(_B, 3 * _N).astype(jnp.float32)
  wflat, wpack = _pack_weights(params)
  out = _sc_call(pos_t, x_t, wflat, wpack)
  return jnp.transpose(out.reshape(_B, 2, _N), (0, 2, 1))


# speedup vs baseline: 1.5105x; 1.5105x over previous
import jax
import jax.numpy as jnp
from jax import lax
from jax.experimental import pallas as pl
from jax.experimental.pallas import tpu as pltpu
from jax.experimental.pallas import tpu_sc as plsc

_POS_HID = 32
_ATTN_HID = 12
_B = 4
_N = 512
_L = 16
_NC = 2
_NS = 16
_TPB = (_NC * _NS) // _B
_PPT = _N // _TPB

_OQKV = 0
_OW1 = 32
_OB1 = 128
_OW2 = 160
_OB2 = 256
_OA1 = 264
_OAB1 = 304
_OA2 = 320
_OAB2 = 360
_PER_LAYER = 368
_OLW = 3 * _PER_LAYER
_OLB = _OLW + 8
_NWP = _OLB + 8

_SEG_OFFS = ([_OQKV, _OW1, _OB1, _OW2, _OB2, _OA1, _OAB1, _OA2, _OAB2],
             [_OLW, _OLB])


def _sc_body(pos_hbm, x_hbm, *rest):
  w_hbm = rest[:29]
  (out_hbm, ptmp, posb, xb, qkvb, idxb, rpb, outb, xnb, fb, wv, wpb, xsh,
   sem_w, sem_x) = rest[29:]
  c = lax.axis_index('c')
  s = lax.axis_index('s')
  bl = s // _TPB
  b = c * 2 + bl
  ch = s % _TPB
  base = ch * _PPT

  iota = lax.iota(jnp.int32, _L)

  wcopies = []
  k = 0
  for l in range(3):
    for off in _SEG_OFFS[0]:
      wcopies.append(pltpu.async_copy(
          w_hbm[k], wpb.at[pl.ds(l * _PER_LAYER + off, w_hbm[k].shape[0])],
          sem_w))
      k += 1
  for off in _SEG_OFFS[1]:
    wcopies.append(pltpu.async_copy(
        w_hbm[k], wpb.at[pl.ds(off, w_hbm[k].shape[0])], sem_w))
    k += 1

  pltpu.sync_copy(pos_hbm.at[b], ptmp)

  iota3 = iota * 3
  for cc in range(3):
    for t in range(_N // _L):
      posb[pl.ds(cc * _N + t * _L, _L)] = plsc.load_gather(
          ptmp, [iota3 + (t * 3 * _L + cc)])

  xcopy = pltpu.async_copy(x_hbm.at[b], ptmp, sem_x)

  def wsp(j):
    return wv[pl.ds(_L * j, _L)]

  wregs = {}

  def wbc(j):
    v = j // _L
    if v not in wregs:
      wregs[v] = wpb[pl.ds(v * _L, _L)]
    return jnp.full((_L,), wregs[v][j % _L], jnp.float32)

  _tgl = [False]

  def wgt(j):
    _tgl[0] = not _tgl[0]
    return wsp(j) if _tgl[0] else wbc(j)

  _Q = 4
  _G = _PPT // _Q

  @plsc.parallel_loop(0, _G)
  def _topk_loop(i):
    pts = []
    for q in range(_Q):
      gq = jnp.full((_L,), base + i + q * _G, jnp.int32)
      pts.append((plsc.load_gather(posb, [gq]),
                  plsc.load_gather(posb, [gq + _N]),
                  plsc.load_gather(posb, [gq + 2 * _N])))

    def chunk_body(j, carry):
      off = j * _L
      cx = posb[pl.ds(off, _L)]
      cy = posb[pl.ds(_N + off, _L)]
      cz = posb[pl.ds(2 * _N + off, _L)]
      ci = iota + off
      out = []
      for q in range(_Q):
        px, py, pz = pts[q]
        bk, bv = carry[2 * q], carry[2 * q + 1]
        dx = px - cx
        dy = py - cy
        dz = pz - cz
        d2 = dx * dx + dy * dy + dz * dz
        ck, cv = plsc.sort_key_val(d2, ci, descending=True)
        keep = bk <= ck
        mk = jnp.where(keep, bk, ck)
        mv = jnp.where(keep, bv, cv)
        nk, nv = plsc.sort_key_val(mk, mv)
        out += [nk, nv]
      return tuple(out)

    inf0 = jnp.full((_L,), jnp.inf, jnp.float32)
    iz = jnp.zeros((_L,), jnp.int32)
    res = lax.fori_loop(0, _N // _L, chunk_body, (inf0, iz) * _Q)

    for q in range(_Q):
      pi = i + q * _G
      px, py, pz = pts[q]
      bv = res[2 * q + 1]
      idxb[pl.ds(pi * _L, _L)] = bv
      gx = plsc.load_gather(posb, [bv])
      gy = plsc.load_gather(posb, [bv + _N])
      gz = plsc.load_gather(posb, [bv + 2 * _N])
      rpb[pl.ds(pi * _L, _L)] = px - gx
      rpb[pl.ds((_PPT + pi) * _L, _L)] = py - gy
      rpb[pl.ds((2 * _PPT + pi) * _L, _L)] = pz - gz

  for cp in wcopies:
    cp.wait()

  @plsc.parallel_loop(0, _NWP // _L)
  def _splat_loop(v):
    w = wpb[pl.ds(v * _L, _L)]
    for lane in range(_L):
      wv[pl.ds((v * _L + lane) * _L, _L)] = jnp.full((_L,), w[lane],
                                                     jnp.float32)

  xcopy.wait()
  for cc in range(3):
    for t in range(_N // _L):
      xb[pl.ds(cc * _N + t * _L, _L)] = plsc.load_gather(
          ptmp, [iota3 + (t * 3 * _L + cc)])

  for l in range(3):
    lb = l * _PER_LAYER

    @plsc.parallel_loop(0, _N // _L)
    def _qkv_loop(j, lb=lb):
      wregs.clear()
      off = j * _L
      x0 = xb[pl.ds(off, _L)]
      x1 = xb[pl.ds(_N + off, _L)]
      x2 = xb[pl.ds(2 * _N + off, _L)]
      for u in range(9):
        qkvb[pl.ds(u * _N + off, _L)] = (x0 * wgt(lb + _OQKV + u) +
                                         x1 * wgt(lb + _OQKV + 9 + u) +
                                         x2 * wgt(lb + _OQKV + 18 + u))

    @plsc.parallel_loop(0, _PPT // 2, unroll=2)
    def _pt_loop(i, lb=lb):
      wregs.clear()
      pts = []
      for pi in (i, i + _PPT // 2):
        gidx = jnp.full((_L,), base + pi, jnp.int32)
        nb = idxb[pl.ds(pi * _L, _L)]
        qx = plsc.load_gather(qkvb, [gidx])
        qy = plsc.load_gather(qkvb, [gidx + _N])
        qz = plsc.load_gather(qkvb, [gidx + 2 * _N])
        kx = plsc.load_gather(qkvb, [nb + 3 * _N])
        ky = plsc.load_gather(qkvb, [nb + 4 * _N])
        kz = plsc.load_gather(qkvb, [nb + 5 * _N])
        vx = plsc.load_gather(qkvb, [nb + 6 * _N])
        vy = plsc.load_gather(qkvb, [nb + 7 * _N])
        vz = plsc.load_gather(qkvb, [nb + 8 * _N])
        rx = rpb[pl.ds(pi * _L, _L)]
        ry = rpb[pl.ds((_PPT + pi) * _L, _L)]
        rz = rpb[pl.ds((2 * _PPT + pi) * _L, _L)]
        pts.append({'pi': pi, 'q': (qx, qy, qz), 'k': (kx, ky, kz),
                    'v': (vx, vy, vz), 'r': (rx, ry, rz)})

      pb = [wgt(lb + _OB2 + cc) for cc in range(3)]
      for p in pts:
        p['pe'] = list(pb)
      for u in range(_POS_HID):
        w1 = [wgt(lb + _OW1 + 32 * cc + u) for cc in range(3)]
        b1 = wgt(lb + _OB1 + u)
        w2 = [wgt(lb + _OW2 + u * 3 + cc) for cc in range(3)]
        for p in pts:
          rx, ry, rz = p['r']
          h = jnp.maximum(rx * w1[0] + ry * w1[1] + rz * w1[2] + b1, 0.0)
          for cc in range(3):
            p['pe'][cc] = p['pe'][cc] + h * w2[cc]

      ab = [wgt(lb + _OAB2 + cc) for cc in range(3)]
      for p in pts:
        p['s'] = [p['q'][cc] - p['k'][cc] + p['pe'][cc] for cc in range(3)]
        p['o'] = list(ab)
      for u in range(_ATTN_HID):
        a1 = [wgt(lb + _OA1 + 12 * cc + u) for cc in range(3)]
        ob1 = wgt(lb + _OAB1 + u)
        a2 = [wgt(lb + _OA2 + u * 3 + cc) for cc in range(3)]
        for p in pts:
          sx, sy, sz = p['s']
          g = jnp.maximum(sx * a1[0] + sy * a1[1] + sz * a1[2] + ob1, 0.0)
          for cc in range(3):
            p['o'][cc] = p['o'][cc] + g * a2[cc]

      for p in pts:
        outs = []
        for cc in range(3):
          e = jnp.exp(p['o'][cc])
          a = e / jnp.sum(e)
          outs.append(jnp.sum(a * (p['v'][cc] + p['pe'][cc])))
        vout = jnp.where(iota == 0, outs[0],
                         jnp.where(iota == 1, outs[1], outs[2]))
        plsc.store_scatter(outb, [jnp.minimum(iota, 2) * _PPT + p['pi']],
                           vout, mask=iota < 3)

    if l < 2:
      for t in range(_PPT // _L):
        for cc in range(3):
          z = outb[pl.ds(cc * _PPT + t * _L, _L)]
          xnb[pl.ds(cc * _PPT + t * _L, _L)] = 1.0 / (1.0 + jnp.exp(-z))
      for cc in range(3):
        pltpu.sync_copy(xnb.at[pl.ds(cc * _PPT, _PPT)],
                        xsh.at[bl, pl.ds(cc * _N + base, _PPT)])
      plsc.subcore_barrier()
      pltpu.sync_copy(xsh.at[bl], xb)
      plsc.subcore_barrier()
    else:
      for t in range(_PPT // _L):
        xs = []
        for cc in range(3):
          z = outb[pl.ds(cc * _PPT + t * _L, _L)]
          xs.append(1.0 / (1.0 + jnp.exp(-z)))
        u0 = (xs[0] * wsp(_OLW + 0) + xs[1] * wsp(_OLW + 2) +
              xs[2] * wsp(_OLW + 4) + wsp(_OLB + 0))
        u1 = (xs[0] * wsp(_OLW + 1) + xs[1] * wsp(_OLW + 3) +
              xs[2] * wsp(_OLW + 5) + wsp(_OLB + 1))
        m = jnp.maximum(u0, u1)
        e0 = jnp.exp(u0 - m)
        e1 = jnp.exp(u1 - m)
        tot = e0 + e1
        plsc.store_scatter(fb, [iota * 2 + t * 2 * _L], e0 / tot)
        plsc.store_scatter(fb, [iota * 2 + t * 2 * _L + 1], e1 / tot)
      pltpu.sync_copy(fb, out_hbm.at[b, pl.ds(base * 2, 2 * _PPT)])


@jax.jit
def _sc_call(pos_r, x_r, *wleaves):
  mesh = plsc.VectorSubcoreMesh(core_axis_name='c', subcore_axis_name='s',
                                num_cores=_NC, num_subcores=_NS)
  return pl.kernel(
      _sc_body,
      out_type=jax.ShapeDtypeStruct((_B, 2 * _N), jnp.float32),
      mesh=mesh,
      scratch_types=[
          pltpu.VMEM((3 * _N,), jnp.float32),
          pltpu.VMEM((3 * _N,), jnp.float32),
          pltpu.VMEM((3 * _N,), jnp.float32),
          pltpu.VMEM((9 * _N,), jnp.float32),
          pltpu.VMEM((_PPT * _L,), jnp.int32),
          pltpu.VMEM((3 * _PPT * _L,), jnp.float32),
          pltpu.VMEM((3 * _PPT,), jnp.float32),
          pltpu.VMEM((3 * _PPT,), jnp.float32),
          pltpu.VMEM((2 * _PPT,), jnp.float32),
          pltpu.VMEM((_NWP * _L,), jnp.float32),
          pltpu.VMEM((_NWP,), jnp.float32),
          pltpu.VMEM_SHARED((2, 3 * _N), jnp.float32),
          pltpu.SemaphoreType.DMA,
          pltpu.SemaphoreType.DMA,
      ],
      compiler_params=pltpu.CompilerParams(use_tc_tiling_on_sc=False,
                                           needs_layout_passes=False),
      name='pt_knn_sc',
  )(pos_r, x_r, *wleaves)


def kernel(feats, pos, mask, params):
  del mask
  pos_r = pos.reshape(_B, 3 * _N)
  x_r = feats.reshape(_B, 3 * _N)
  leaves = []
  for ln in ('l1', 'l2', 'l3'):
    p = params[ln]
    leaves += [p['qkv'].reshape(-1), p['pos_w1'].reshape(-1), p['pos_b1'],
               p['pos_w2'].reshape(-1), p['pos_b2'], p['attn_w1'].reshape(-1),
               p['attn_b1'], p['attn_w2'].reshape(-1), p['attn_b2']]
  leaves += [params['lin_w'].reshape(-1), params['lin_b']]
  out = _sc_call(pos_r, x_r, *leaves)
  return out.reshape(_B, _N, 2)


# speedup vs baseline: 1.5849x; 1.0493x over previous
import jax
import jax.numpy as jnp
from jax import lax
from jax.experimental import pallas as pl
from jax.experimental.pallas import tpu as pltpu
from jax.experimental.pallas import tpu_sc as plsc

_POS_HID = 32
_ATTN_HID = 12
_B = 4
_N = 512
_L = 16
_NC = 2
_NS = 16
_TPB = (_NC * _NS) // _B
_PPT = _N // _TPB

_OQKV = 0
_OW1 = 27
_OB1 = 123
_OW2 = 155
_OB2 = 251
_OA1 = 254
_OAB1 = 290
_OA2 = 302
_OAB2 = 338
_PER_LAYER = 341
_OLW = 3 * _PER_LAYER
_OLB = _OLW + 6
_NW = _OLB + 2
_NWPAD = ((_NW + _L - 1) // _L) * _L


def _pack_weights(params):
  segs = []
  for ln in ('l1', 'l2', 'l3'):
    p = params[ln]
    segs += [p['qkv'].reshape(-1), p['pos_w1'].reshape(-1), p['pos_b1'],
             p['pos_w2'].reshape(-1), p['pos_b2'], p['attn_w1'].reshape(-1),
             p['attn_b1'], p['attn_w2'].reshape(-1), p['attn_b2']]
  segs += [params['lin_w'].reshape(-1), params['lin_b'],
           jnp.zeros((_NWPAD - _NW,), jnp.float32)]
  w = jnp.concatenate(segs).astype(jnp.float32)
  return jnp.repeat(w[:_NW], _L), w


def _sc_body(pos_hbm, x_hbm, w_hbm, wp_hbm, out_hbm,
             posb, xb, qkvb, idxb, rpb, outb, xnb, fb, wv, wpb, xsh):
  c = lax.axis_index('c')
  s = lax.axis_index('s')
  bl = s // _TPB
  b = c * 2 + bl
  ch = s % _TPB
  base = ch * _PPT

  pltpu.sync_copy(pos_hbm.at[b], posb)
  pltpu.sync_copy(x_hbm.at[b], xb)
  pltpu.sync_copy(w_hbm, wv)
  pltpu.sync_copy(wp_hbm, wpb)

  iota = lax.iota(jnp.int32, _L)

  def wsp(j):
    return wv[pl.ds(_L * j, _L)]

  wregs = {}

  def wbc(j):
    v = j // _L
    if v not in wregs:
      wregs[v] = wpb[pl.ds(v * _L, _L)]
    return jnp.full((_L,), wregs[v][j % _L], jnp.float32)

  _tgl = [False]

  def wgt(j):
    _tgl[0] = not _tgl[0]
    return wsp(j) if _tgl[0] else wbc(j)

  _Q = 4
  _G = _PPT // _Q

  @plsc.parallel_loop(0, _G)
  def _topk_loop(i):
    pts = []
    for q in range(_Q):
      gq = jnp.full((_L,), base + i + q * _G, jnp.int32)
      pts.append((plsc.load_gather(posb, [gq]),
                  plsc.load_gather(posb, [gq + _N]),
                  plsc.load_gather(posb, [gq + 2 * _N])))

    def chunk_body(j, carry):
      off = j * _L
      cx = posb[pl.ds(off, _L)]
      cy = posb[pl.ds(_N + off, _L)]
      cz = posb[pl.ds(2 * _N + off, _L)]
      ci = iota + off
      out = []
      for q in range(_Q):
        px, py, pz = pts[q]
        bk, bv = carry[2 * q], carry[2 * q + 1]
        dx = px - cx
        dy = py - cy
        dz = pz - cz
        d2 = dx * dx + dy * dy + dz * dz
        ck, cv = plsc.sort_key_val(d2, ci, descending=True)
        keep = bk <= ck
        mk = jnp.where(keep, bk, ck)
        mv = jnp.where(keep, bv, cv)
        nk, nv = plsc.sort_key_val(mk, mv)
        out += [nk, nv]
      return tuple(out)

    inf0 = jnp.full((_L,), jnp.inf, jnp.float32)
    iz = jnp.zeros((_L,), jnp.int32)
    res = lax.fori_loop(0, _N // _L, chunk_body, (inf0, iz) * _Q)

    for q in range(_Q):
      pi = i + q * _G
      px, py, pz = pts[q]
      bv = res[2 * q + 1]
      idxb[pl.ds(pi * _L, _L)] = bv
      gx = plsc.load_gather(posb, [bv])
      gy = plsc.load_gather(posb, [bv + _N])
      gz = plsc.load_gather(posb, [bv + 2 * _N])
      rpb[pl.ds(pi * _L, _L)] = px - gx
      rpb[pl.ds((_PPT + pi) * _L, _L)] = py - gy
      rpb[pl.ds((2 * _PPT + pi) * _L, _L)] = pz - gz

  for l in range(3):
    lb = l * _PER_LAYER

    @plsc.parallel_loop(0, _N // _L)
    def _qkv_loop(j, lb=lb):
      wregs.clear()
      off = j * _L
      x0 = xb[pl.ds(off, _L)]
      x1 = xb[pl.ds(_N + off, _L)]
      x2 = xb[pl.ds(2 * _N + off, _L)]
      for u in range(9):
        qkvb[pl.ds(u * _N + off, _L)] = (x0 * wgt(lb + _OQKV + u) +
                                         x1 * wgt(lb + _OQKV + 9 + u) +
                                         x2 * wgt(lb + _OQKV + 18 + u))

    @plsc.parallel_loop(0, _PPT // 2, unroll=2)
    def _pt_loop(i, lb=lb):
      wregs.clear()
      pts = []
      for pi in (i, i + _PPT // 2):
        gidx = jnp.full((_L,), base + pi, jnp.int32)
        nb = idxb[pl.ds(pi * _L, _L)]
        qx = plsc.load_gather(qkvb, [gidx])
        qy = plsc.load_gather(qkvb, [gidx + _N])
        qz = plsc.load_gather(qkvb, [gidx + 2 * _N])
        kx = plsc.load_gather(qkvb, [nb + 3 * _N])
        ky = plsc.load_gather(qkvb, [nb + 4 * _N])
        kz = plsc.load_gather(qkvb, [nb + 5 * _N])
        vx = plsc.load_gather(qkvb, [nb + 6 * _N])
        vy = plsc.load_gather(qkvb, [nb + 7 * _N])
        vz = plsc.load_gather(qkvb, [nb + 8 * _N])
        rx = rpb[pl.ds(pi * _L, _L)]
        ry = rpb[pl.ds((_PPT + pi) * _L, _L)]
        rz = rpb[pl.ds((2 * _PPT + pi) * _L, _L)]
        pts.append({'pi': pi, 'q': (qx, qy, qz), 'k': (kx, ky, kz),
                    'v': (vx, vy, vz), 'r': (rx, ry, rz)})

      pb = [wgt(lb + _OB2 + cc) for cc in range(3)]
      zv = jnp.zeros((_L,), jnp.float32)
      for p in pts:
        p['pe'] = list(pb)
        p['pe2'] = [zv, zv, zv]
      for u in range(_POS_HID):
        w1 = [wgt(lb + _OW1 + 32 * cc + u) for cc in range(3)]
        b1 = wgt(lb + _OB1 + u)
        w2 = [wgt(lb + _OW2 + u * 3 + cc) for cc in range(3)]
        acc = 'pe' if u % 2 == 0 else 'pe2'
        for p in pts:
          rx, ry, rz = p['r']
          h = jnp.maximum(rx * w1[0] + ry * w1[1] + rz * w1[2] + b1, 0.0)
          for cc in range(3):
            p[acc][cc] = p[acc][cc] + h * w2[cc]
      for p in pts:
        p['pe'] = [p['pe'][cc] + p['pe2'][cc] for cc in range(3)]

      ab = [wgt(lb + _OAB2 + cc) for cc in range(3)]
      for p in pts:
        p['s'] = [p['q'][cc] - p['k'][cc] + p['pe'][cc] for cc in range(3)]
        p['o'] = list(ab)
        p['o2'] = [zv, zv, zv]
      for u in range(_ATTN_HID):
        a1 = [wgt(lb + _OA1 + 12 * cc + u) for cc in range(3)]
        ob1 = wgt(lb + _OAB1 + u)
        a2 = [wgt(lb + _OA2 + u * 3 + cc) for cc in range(3)]
        acc = 'o' if u % 2 == 0 else 'o2'
        for p in pts:
          sx, sy, sz = p['s']
          g = jnp.maximum(sx * a1[0] + sy * a1[1] + sz * a1[2] + ob1, 0.0)
          for cc in range(3):
            p[acc][cc] = p[acc][cc] + g * a2[cc]
      for p in pts:
        p['o'] = [p['o'][cc] + p['o2'][cc] for cc in range(3)]

      for p in pts:
        outs = []
        for cc in range(3):
          e = jnp.exp(p['o'][cc])
          a = e / jnp.sum(e)
          outs.append(jnp.sum(a * (p['v'][cc] + p['pe'][cc])))
        vout = jnp.where(iota == 0, outs[0],
                         jnp.where(iota == 1, outs[1], outs[2]))
        plsc.store_scatter(outb, [jnp.minimum(iota, 2) * _PPT + p['pi']],
                           vout, mask=iota < 3)

    if l < 2:
      for t in range(_PPT // _L):
        for cc in range(3):
          z = outb[pl.ds(cc * _PPT + t * _L, _L)]
          xnb[pl.ds(cc * _PPT + t * _L, _L)] = 1.0 / (1.0 + jnp.exp(-z))
      for cc in range(3):
        pltpu.sync_copy(xnb.at[pl.ds(cc * _PPT, _PPT)],
                        xsh.at[bl, pl.ds(cc * _N + base, _PPT)])
      plsc.subcore_barrier()
      pltpu.sync_copy(xsh.at[bl], xb)
      plsc.subcore_barrier()
    else:
      for t in range(_PPT // _L):
        xs = []
        for cc in range(3):
          z = outb[pl.ds(cc * _PPT + t * _L, _L)]
          xs.append(1.0 / (1.0 + jnp.exp(-z)))
        u0 = (xs[0] * wsp(_OLW + 0) + xs[1] * wsp(_OLW + 2) +
              xs[2] * wsp(_OLW + 4) + wsp(_OLB + 0))
        u1 = (xs[0] * wsp(_OLW + 1) + xs[1] * wsp(_OLW + 3) +
              xs[2] * wsp(_OLW + 5) + wsp(_OLB + 1))
        m = jnp.maximum(u0, u1)
        e0 = jnp.exp(u0 - m)
        e1 = jnp.exp(u1 - m)
        tot = e0 + e1
        fb[pl.ds(t * _L, _L)] = e0 / tot
        fb[pl.ds(_PPT + t * _L, _L)] = e1 / tot
      for o in range(2):
        pltpu.sync_copy(fb.at[pl.ds(o * _PPT, _PPT)],
                        out_hbm.at[b, pl.ds(o * _N + base, _PPT)])


@jax.jit
def _sc_call(pos_t, x_t, wflat, wpack):
  mesh = plsc.VectorSubcoreMesh(core_axis_name='c', subcore_axis_name='s',
                                num_cores=_NC, num_subcores=_NS)
  return pl.kernel(
      _sc_body,
      out_type=jax.ShapeDtypeStruct((_B, 2 * _N), jnp.float32),
      mesh=mesh,
      scratch_types=[
          pltpu.VMEM((3 * _N,), jnp.float32),
          pltpu.VMEM((3 * _N,), jnp.float32),
          pltpu.VMEM((9 * _N,), jnp.float32),
          pltpu.VMEM((_PPT * _L,), jnp.int32),
          pltpu.VMEM((3 * _PPT * _L,), jnp.float32),
          pltpu.VMEM((3 * _PPT,), jnp.float32),
          pltpu.VMEM((3 * _PPT,), jnp.float32),
          pltpu.VMEM((2 * _PPT,), jnp.float32),
          pltpu.VMEM((_NW * _L,), jnp.float32),
          pltpu.VMEM((_NWPAD,), jnp.float32),
          pltpu.VMEM_SHARED((2, 3 * _N), jnp.float32),
      ],
      compiler_params=pltpu.CompilerParams(use_tc_tiling_on_sc=False,
                                           needs_layout_passes=False),
      name='pt_knn_sc',
  )(pos_t, x_t, wflat, wpack)


def kernel(feats, pos, mask, params):
  del mask
  pos_t = jnp.transpose(pos, (0, 2, 1)).reshape(_B, 3 * _N).astype(jnp.float32)
  x_t = jnp.transpose(feats, (0, 2, 1)).reshape(_B, 3 * _N).astype(jnp.float32)
  wflat, wpack = _pack_weights(params)
  out = _sc_call(pos_t, x_t, wflat, wpack)
  return jnp.transpose(out.reshape(_B, 2, _N), (0, 2, 1))
